# baseline (speedup 1.0000x reference)
import functools

import jax
import jax.numpy as jnp
from jax import lax
from jax.experimental import pallas as pl
from jax.experimental.pallas import tpu as pltpu
from jax.experimental.pallas import tpu_sc as plsc

R, S, D = 16384, 200, 32
NW = 32
ROWS_PW = R // NW
CR = 8
NCHUNK = ROWS_PW // CR
SPLITS = ((0, 128), (128, 72))
CHUNK_BYTES = CR * S * D * 4

_mesh = plsc.VectorSubcoreMesh(core_axis_name="c", subcore_axis_name="s")


@functools.partial(
    pl.kernel,
    mesh=_mesh,
    out_type=jax.ShapeDtypeStruct((R, S, D), jnp.float32),
    scratch_types=[
        pltpu.VMEM((2, CR, S), jnp.int32),
        pltpu.VMEM((2, CR, S, D), jnp.float32),
        pltpu.SemaphoreType.DMA((2,)),
        pltpu.SemaphoreType.DMA((2,)),
    ],
    compiler_params=pltpu.CompilerParams(use_tc_tiling_on_sc=False),
)
def _sc_gather(table_hbm, idx_hbm, out_hbm, idx_v, rows_v, gsem, ssem):
    wid = lax.axis_index("s") * 2 + lax.axis_index("c")
    row0 = wid * ROWS_PW

    def fire_chunk(c, b):
        base_row = row0 + c * CR
        pltpu.sync_copy(idx_hbm.at[pl.ds(base_row, CR)], idx_v.at[b])
        for i in range(CR):
            for off, ln in SPLITS:
                pltpu.async_copy(
                    table_hbm.at[idx_v.at[b].at[i].at[pl.ds(off, ln)]],
                    rows_v.at[b].at[i].at[pl.ds(off, ln)],
                    gsem.at[b],
                )

    def wait_gathers(b):
        pltpu.make_async_copy(
            out_hbm.at[pl.ds(0, CR)], rows_v.at[b], gsem.at[b]
        ).wait()

    def fire_store(c, b):
        base_row = row0 + c * CR
        pltpu.async_copy(
            rows_v.at[b],
            out_hbm.at[pl.ds(base_row, CR)],
            ssem.at[b],
        )

    def wait_store(b):
        pltpu.make_async_copy(
            rows_v.at[b], out_hbm.at[pl.ds(0, CR)], ssem.at[b]
        ).wait()

    fire_chunk(0, 0)
    fire_chunk(1, 1)
    wait_gathers(0)
    fire_store(0, 0)

    def body(g, _):
        for b in range(2):
            c = 2 + 2 * g + b
            wait_store(b)
            fire_chunk(c, b)
            wait_gathers(1 - b)
            fire_store(c - 1, 1 - b)
        return ()

    lax.fori_loop(0, (NCHUNK - 2) // 2, body, (), unroll=False)

    wait_gathers(1)
    fire_store(NCHUNK - 1, 1)
    wait_store(0)
    wait_store(1)


def kernel(visit_segments, embedding_weight):
    return _sc_gather(embedding_weight, visit_segments.astype(jnp.int32))


# speedup vs baseline: 1.0150x; 1.0150x over previous
import functools

import jax
import jax.numpy as jnp
from jax import lax
from jax.experimental import pallas as pl
from jax.experimental.pallas import tpu as pltpu
from jax.experimental.pallas import tpu_sc as plsc

R, S, D = 16384, 200, 32
V = 1000
L = 16
RB = R // 128
NU = S * RB
NW = 32
UPW = NU // NW

_mesh = plsc.VectorSubcoreMesh(core_axis_name="c", subcore_axis_name="s")


@functools.partial(
    pl.kernel,
    mesh=_mesh,
    out_type=jax.ShapeDtypeStruct((S, D // 8, RB, 8, 128), jnp.float32),
    scratch_types=[
        pltpu.VMEM((V * D,), jnp.float32),
        pltpu.VMEM((2, 128), jnp.int32),
        pltpu.VMEM((2, D, 128), jnp.float32),
        pltpu.SemaphoreType.DMA((2,)),
        pltpu.SemaphoreType.DMA((2,)),
    ],
    compiler_params=pltpu.CompilerParams(
        use_tc_tiling_on_sc=False, needs_layout_passes=False
    ),
)
def _sc_lookup(table_hbm, idx_hbm, out_hbm, table_v, idx_v, blk_v, isem, ssem):
    wid = lax.axis_index("s") * 2 + lax.axis_index("c")
    u0 = wid * UPW

    pltpu.sync_copy(table_hbm, table_v)

    def coords(u):
        s = u // 128
        rb = lax.rem(u, 128)
        return s // 8, lax.rem(s, 8), s, rb

    def fire_idx(u, b):
        sb, si, _, rb = coords(lax.min(u, NU - 1))
        pltpu.async_copy(idx_hbm.at[sb, rb, si], idx_v.at[b], isem.at[b])

    def wait_idx(b):
        pltpu.make_async_copy(
            idx_hbm.at[0, 0, 0], idx_v.at[b], isem.at[b]
        ).wait()

    def compute(b):
        for v in range(8):
            iv = idx_v[b, pl.ds(v * L, L)]
            fm = iv * D
            for d in range(D):
                g = plsc.load_gather(table_v, [fm + d])
                blk_v[b, d, pl.ds(v * L, L)] = g

    def fire_store(u, b):
        _, _, s, rb = coords(u)
        for db in range(D // 8):
            pltpu.async_copy(
                blk_v.at[b].at[pl.ds(db * 8, 8)],
                out_hbm.at[s, db, rb],
                ssem.at[b],
            )

    def wait_store(b):
        for db in range(D // 8):
            pltpu.make_async_copy(
                blk_v.at[b].at[pl.ds(db * 8, 8)],
                out_hbm.at[0, 0, 0],
                ssem.at[b],
            ).wait()

    fire_idx(u0, 0)
    fire_idx(u0 + 1, 1)
    for b in range(2):
        wait_idx(b)
        compute(b)
        fire_store(u0 + b, b)
        fire_idx(u0 + b + 2, b)

    def body(g, _):
        for b in range(2):
            u = u0 + 2 + 2 * g + b
            wait_idx(b)
            wait_store(b)
            compute(b)
            fire_store(u, b)
            fire_idx(u + 2, b)
        return ()

    lax.fori_loop(0, (UPW - 2) // 2, body, (), unroll=False)

    for b in range(2):
        wait_idx(b)
        wait_store(b)


def kernel(visit_segments, embedding_weight):
    idx_t = (
        visit_segments.astype(jnp.int32)
        .reshape(128, 128, 25, 8)
        .transpose(2, 0, 3, 1)
    )
    out_t = _sc_lookup(embedding_weight.reshape(-1), idx_t)
    return out_t.transpose(2, 4, 0, 1, 3).reshape(R, S, D)


# speedup vs baseline: 1.5673x; 1.5441x over previous
import functools

import jax
import jax.numpy as jnp
from jax import lax
from jax.experimental import pallas as pl
from jax.experimental.pallas import tpu as pltpu
from jax.experimental.pallas import tpu_sc as plsc

R, S, D = 16384, 200, 32
V = 1000
L = 16
RB = R // 128
NU = S * RB
NW = 32
UPW = NU // NW

_mesh = plsc.VectorSubcoreMesh(core_axis_name="c", subcore_axis_name="s")


@functools.partial(
    pl.kernel,
    mesh=_mesh,
    out_type=jax.ShapeDtypeStruct((S, D // 8, RB, 8, 128), jnp.float32),
    scratch_types=[
        pltpu.VMEM((V * D,), jnp.float32),
        pltpu.VMEM((2, 128), jnp.int32),
        pltpu.VMEM((2, D, 128), jnp.float32),
        pltpu.SemaphoreType.DMA((2,)),
        pltpu.SemaphoreType.DMA((2,)),
    ],
    compiler_params=pltpu.CompilerParams(
        use_tc_tiling_on_sc=False, needs_layout_passes=False
    ),
)
def _sc_lookup(table_hbm, idx_hbm, out_hbm, table_v, idx_v, blk_v, isem, ssem):
    wid = lax.axis_index("s") * 2 + lax.axis_index("c")
    u0 = wid * UPW

    pltpu.sync_copy(table_hbm, table_v)

    def coords(u):
        s = u // 128
        rb = lax.rem(u, 128)
        return s // 8, lax.rem(s, 8), s, rb

    def fire_idx(u, b):
        sb, si, _, rb = coords(lax.min(u, NU - 1))
        pltpu.async_copy(idx_hbm.at[sb, rb, si], idx_v.at[b], isem.at[b])

    def wait_idx(b):
        pltpu.make_async_copy(
            idx_hbm.at[0, 0, 0], idx_v.at[b], isem.at[b]
        ).wait()

    def compute(b):
        @plsc.parallel_loop(0, 128 // L, unroll=2)
        def _(v):
            vs = pl.multiple_of(v * L, L)
            iv = idx_v[b, pl.ds(vs, L)]
            fm = iv * D
            for d in range(D):
                g = plsc.load_gather(table_v, [fm + d])
                blk_v[b, d, pl.ds(vs, L)] = g

    def fire_store(u, b):
        _, _, s, rb = coords(u)
        for db in range(D // 8):
            pltpu.async_copy(
                blk_v.at[b].at[pl.ds(db * 8, 8)],
                out_hbm.at[s, db, rb],
                ssem.at[b],
            )

    def wait_store(b):
        for db in range(D // 8):
            pltpu.make_async_copy(
                blk_v.at[b].at[pl.ds(db * 8, 8)],
                out_hbm.at[0, 0, 0],
                ssem.at[b],
            ).wait()

    fire_idx(u0, 0)
    fire_idx(u0 + 1, 1)
    for b in range(2):
        wait_idx(b)
        compute(b)
        fire_store(u0 + b, b)
        fire_idx(u0 + b + 2, b)

    def body(g, _):
        for b in range(2):
            u = u0 + 2 + 2 * g + b
            wait_idx(b)
            wait_store(b)
            compute(b)
            fire_store(u, b)
            fire_idx(u + 2, b)
        return ()

    lax.fori_loop(0, (UPW - 2) // 2, body, (), unroll=False)

    for b in range(2):
        wait_idx(b)
        wait_store(b)


def kernel(visit_segments, embedding_weight):
    idx_t = (
        visit_segments.astype(jnp.int32)
        .reshape(128, 128, 25, 8)
        .transpose(2, 0, 3, 1)
    )
    out_t = _sc_lookup(embedding_weight.reshape(-1), idx_t)
    return out_t.transpose(2, 4, 0, 1, 3).reshape(R, S, D)


# speedup vs baseline: 6.4221x; 4.0977x over previous
import functools

import jax
import jax.numpy as jnp
from jax import lax
from jax.experimental import pallas as pl
from jax.experimental.pallas import tpu as pltpu
from jax.experimental.pallas import tpu_sc as plsc

R, S, D = 16384, 200, 32
V = 1000
L = 16
RB = R // 128
NU = S * RB
NW = 32
UPW = NU // NW

_mesh = plsc.VectorSubcoreMesh(core_axis_name="c", subcore_axis_name="s")


@functools.partial(
    pl.kernel,
    mesh=_mesh,
    out_type=jax.ShapeDtypeStruct((S, D // 8, RB, 8, 128), jnp.float32),
    scratch_types=[
        pltpu.VMEM((V * (D + 1),), jnp.float32),
        pltpu.VMEM((2, 128), jnp.int32),
        pltpu.VMEM((2, D, 128), jnp.float32),
        pltpu.SemaphoreType.DMA((2,)),
        pltpu.SemaphoreType.DMA((2,)),
    ],
    compiler_params=pltpu.CompilerParams(
        use_tc_tiling_on_sc=False, needs_layout_passes=False
    ),
)
def _sc_lookup(table_hbm, idx_hbm, out_hbm, table_v, idx_v, blk_v, isem, ssem):
    wid = lax.axis_index("s") * 2 + lax.axis_index("c")
    u0 = wid * UPW

    pltpu.sync_copy(table_hbm, table_v)

    def coords(u):
        s = u // 128
        rb = lax.rem(u, 128)
        return s // 8, lax.rem(s, 8), s, rb

    def fire_idx(u, b):
        sb, si, _, rb = coords(lax.min(u, NU - 1))
        pltpu.async_copy(idx_hbm.at[sb, rb, si], idx_v.at[b], isem.at[b])

    def wait_idx(b):
        pltpu.make_async_copy(
            idx_hbm.at[0, 0, 0], idx_v.at[b], isem.at[b]
        ).wait()

    def compute(b):
        @plsc.parallel_loop(0, 128 // L, unroll=2)
        def _(v):
            vs = pl.multiple_of(v * L, L)
            iv = idx_v[b, pl.ds(vs, L)]
            fm = iv * (D + 1)
            for d in range(D):
                g = plsc.load_gather(table_v, [fm + d])
                blk_v[b, d, pl.ds(vs, L)] = g

    def fire_store(u, b):
        _, _, s, rb = coords(u)
        for db in range(D // 8):
            pltpu.async_copy(
                blk_v.at[b].at[pl.ds(db * 8, 8)],
                out_hbm.at[s, db, rb],
                ssem.at[b],
            )

    def wait_store(b):
        for db in range(D // 8):
            pltpu.make_async_copy(
                blk_v.at[b].at[pl.ds(db * 8, 8)],
                out_hbm.at[0, 0, 0],
                ssem.at[b],
            ).wait()

    fire_idx(u0, 0)
    fire_idx(u0 + 1, 1)
    for b in range(2):
        wait_idx(b)
        compute(b)
        fire_store(u0 + b, b)
        fire_idx(u0 + b + 2, b)

    def body(g, _):
        for b in range(2):
            u = u0 + 2 + 2 * g + b
            wait_idx(b)
            wait_store(b)
            compute(b)
            fire_store(u, b)
            fire_idx(u + 2, b)
        return ()

    lax.fori_loop(0, (UPW - 2) // 2, body, (), unroll=False)

    for b in range(2):
        wait_idx(b)
        wait_store(b)


def kernel(visit_segments, embedding_weight):
    idx_t = (
        visit_segments.astype(jnp.int32)
        .reshape(128, 128, 25, 8)
        .transpose(2, 0, 3, 1)
    )
    table_pad = jnp.pad(embedding_weight, ((0, 0), (0, 1))).reshape(-1)
    out_t = _sc_lookup(table_pad, idx_t)
    return out_t.transpose(2, 4, 0, 1, 3).reshape(R, S, D)


# speedup vs baseline: 6.6062x; 1.0287x over previous
import functools

import jax
import jax.numpy as jnp
from jax import lax
from jax.experimental import pallas as pl
from jax.experimental.pallas import tpu as pltpu
from jax.experimental.pallas import tpu_sc as plsc

R, S, D = 16384, 200, 32
V = 1000
L = 16
RB = R // 128
NU = S * RB
NW = 32
UPW = NU // NW

_mesh = plsc.VectorSubcoreMesh(core_axis_name="c", subcore_axis_name="s")


@functools.partial(
    pl.kernel,
    mesh=_mesh,
    out_type=jax.ShapeDtypeStruct((S, D // 8, RB, 8, 128), jnp.float32),
    scratch_types=[
        pltpu.VMEM((V * (D + 1),), jnp.float32),
        pltpu.VMEM((2, 128), jnp.int32),
        pltpu.VMEM((2, D, 128), jnp.float32),
        pltpu.SemaphoreType.DMA((2,)),
        pltpu.SemaphoreType.DMA((2,)),
    ],
    compiler_params=pltpu.CompilerParams(
        use_tc_tiling_on_sc=False, needs_layout_passes=False
    ),
)
def _sc_lookup(table_hbm, idx_hbm, out_hbm, table_v, idx_v, blk_v, isem, ssem):
    wid = lax.axis_index("s") * 2 + lax.axis_index("c")
    u0 = wid * UPW

    pltpu.sync_copy(table_hbm, table_v)

    def coords(u):
        s = u // 128
        rb = lax.rem(u, 128)
        return s // 8, lax.rem(s, 8), s, rb

    def fire_idx(u, b):
        sb, si, _, rb = coords(lax.min(u, NU - 1))
        pltpu.async_copy(idx_hbm.at[sb, rb, si], idx_v.at[b], isem.at[b])

    def wait_idx(b):
        pltpu.make_async_copy(
            idx_hbm.at[0, 0, 0], idx_v.at[b], isem.at[b]
        ).wait()

    def compute(b):
        @plsc.parallel_loop(0, 128 // L, unroll=4)
        def _(v):
            vs = pl.multiple_of(v * L, L)
            iv = idx_v[b, pl.ds(vs, L)]
            fm = iv * (D + 1)
            for d in range(D):
                g = plsc.load_gather(table_v, [fm + d])
                blk_v[b, d, pl.ds(vs, L)] = g

    def fire_store(u, b):
        _, _, s, rb = coords(u)
        for db in range(D // 8):
            pltpu.async_copy(
                blk_v.at[b].at[pl.ds(db * 8, 8)],
                out_hbm.at[s, db, rb],
                ssem.at[b],
            )

    def wait_store(b):
        for db in range(D // 8):
            pltpu.make_async_copy(
                blk_v.at[b].at[pl.ds(db * 8, 8)],
                out_hbm.at[0, 0, 0],
                ssem.at[b],
            ).wait()

    fire_idx(u0, 0)
    fire_idx(u0 + 1, 1)
    for b in range(2):
        wait_idx(b)
        compute(b)
        fire_store(u0 + b, b)
        fire_idx(u0 + b + 2, b)

    def body(g, _):
        for b in range(2):
            u = u0 + 2 + 2 * g + b
            wait_idx(b)
            wait_store(b)
            compute(b)
            fire_store(u, b)
            fire_idx(u + 2, b)
        return ()

    lax.fori_loop(0, (UPW - 2) // 2, body, (), unroll=False)

    for b in range(2):
        wait_idx(b)
        wait_store(b)


def kernel(visit_segments, embedding_weight):
    idx_t = (
        visit_segments.astype(jnp.int32)
        .reshape(128, 128, 25, 8)
        .transpose(2, 0, 3, 1)
    )
    table_pad = jnp.pad(embedding_weight, ((0, 0), (0, 1))).reshape(-1)
    out_t = _sc_lookup(table_pad, idx_t)
    return out_t.transpose(2, 4, 0, 1, 3).reshape(R, S, D)


# speedup vs baseline: 8.4088x; 1.2729x over previous
import functools

import jax
import jax.numpy as jnp
from jax import lax
from jax.experimental import pallas as pl
from jax.experimental.pallas import tpu as pltpu
from jax.experimental.pallas import tpu_sc as plsc

R, S, D = 16384, 200, 32
V = 1000
L = 16
RB = R // 128
NU = S * RB
NW = 32
UPW = NU // NW

_mesh = plsc.VectorSubcoreMesh(core_axis_name="c", subcore_axis_name="s")


@functools.partial(
    pl.kernel,
    mesh=_mesh,
    out_type=jax.ShapeDtypeStruct((S, D // 8, RB, 8, 128), jnp.float32),
    scratch_types=[
        pltpu.VMEM((V * (D + 1),), jnp.float32),
        pltpu.VMEM((4, 128), jnp.int32),
        pltpu.VMEM((4, D, 128), jnp.float32),
        pltpu.SemaphoreType.DMA((4,)),
        pltpu.SemaphoreType.DMA((4,)),
    ],
    compiler_params=pltpu.CompilerParams(
        use_tc_tiling_on_sc=False, needs_layout_passes=False
    ),
)
def _sc_lookup(table_hbm, idx_hbm, out_hbm, table_v, idx_v, blk_v, isem, ssem):
    wid = lax.axis_index("s") * 2 + lax.axis_index("c")
    u0 = wid * UPW

    pltpu.sync_copy(table_hbm, table_v)

    def coords(u):
        s = u // 128
        rb = lax.rem(u, 128)
        return s // 8, lax.rem(s, 8), s, rb

    def fire_idx(u, b):
        sb, si, _, rb = coords(lax.min(u, NU - 1))
        pltpu.async_copy(idx_hbm.at[sb, rb, si], idx_v.at[b], isem.at[b])

    def wait_idx(b):
        pltpu.make_async_copy(
            idx_hbm.at[0, 0, 0], idx_v.at[b], isem.at[b]
        ).wait()

    def compute(b):
        @plsc.parallel_loop(0, 128 // L, unroll=4)
        def _(v):
            vs = pl.multiple_of(v * L, L)
            iv = idx_v[b, pl.ds(vs, L)]
            fm = iv * (D + 1)
            for d in range(D):
                g = plsc.load_gather(table_v, [fm + d])
                blk_v[b, d, pl.ds(vs, L)] = g

    def fire_store(u, b):
        _, _, s, rb = coords(u)
        for db in range(D // 8):
            pltpu.async_copy(
                blk_v.at[b].at[pl.ds(db * 8, 8)],
                out_hbm.at[s, db, rb],
                ssem.at[b],
            )

    def wait_store(b):
        for db in range(D // 8):
            pltpu.make_async_copy(
                blk_v.at[b].at[pl.ds(db * 8, 8)],
                out_hbm.at[0, 0, 0],
                ssem.at[b],
            ).wait()

    NB = 4
    for b in range(NB):
        fire_idx(u0 + b, b)
    for b in range(NB):
        wait_idx(b)
        compute(b)
        fire_store(u0 + b, b)
        fire_idx(u0 + b + NB, b)

    def body(g, _):
        for b in range(NB):
            u = u0 + NB + NB * g + b
            wait_idx(b)
            wait_store(b)
            compute(b)
            fire_store(u, b)
            fire_idx(u + NB, b)
        return ()

    lax.fori_loop(0, (UPW - NB) // NB, body, (), unroll=False)

    for b in range(NB):
        wait_idx(b)
        wait_store(b)


def kernel(visit_segments, embedding_weight):
    idx_t = (
        visit_segments.astype(jnp.int32)
        .reshape(128, 128, 25, 8)
        .transpose(2, 0, 3, 1)
    )
    table_pad = jnp.pad(embedding_weight, ((0, 0), (0, 1))).reshape(-1)
    out_t = _sc_lookup(table_pad, idx_t)
    return out_t.transpose(2, 4, 0, 1, 3).reshape(R, S, D)
